# baseline (speedup 1.0000x reference)
import jax
import jax.numpy as jnp
from jax import lax
from jax.experimental import pallas as pl
from jax.experimental.pallas import tpu as pltpu
from jax.experimental.pallas import tpu_sc as plsc

_B = 64
_PERIODS = 128
_SEG_A = 48
_SEG_B = 80
_NW = 32
_TILES = 3 * (_B // 8) * _PERIODS
_CHUNK_T = 8
_CHUNKS = _TILES // _CHUNK_T
_CHUNKS_PER_W = _CHUNKS // _NW


def _sc_body(x_hbm, o_hbm, buf0, buf1, outv, sem0, sem1, osem):
    cid = lax.axis_index("c")
    sid = lax.axis_index("s")
    w = sid * 2 + cid

    iota = lax.iota(jnp.int32, 16)
    row_idx = jnp.remainder(iota, 8)
    pair_out = (iota // 8) * 2
    comb = [(iota // 8) * 128 + ((iota + u) & 15) for u in range(16)]
    zero = jnp.zeros((16,), jnp.float32)
    inv_a = jnp.float32(1.0 / _SEG_A)
    inv_b = jnp.float32(1.0 / _SEG_B)

    bufs = (buf0, buf1)
    sems = (sem0, sem1)

    def chunk_coords(j):
        kg = w * _CHUNKS_PER_W + j
        c = kg // 128
        rem = kg - c * 128
        tr = rem // 16
        cb = rem - tr * 16
        return c, tr, cb

    def chunk_slice(j):
        c, tr, cb = chunk_coords(j)
        return x_hbm.at[c, pl.ds(tr * 8, 8), pl.ds(cb * 1024, 1024)]

    def compute(buf, j):
        jvec = jnp.full((16,), j, jnp.int32)

        def pair(tp, carry):
            base = tp * 256

            def seg_sum(start, n_atoms):
                even = odd = zero
                for k in range(n_atoms // 16):
                    off = base + start + 16 * k
                    for u in range(0, 16, 2):
                        even = even + plsc.load_gather(
                            buf, [row_idx, comb[u] + off])
                        odd = odd + plsc.load_gather(
                            buf, [row_idx, comb[u + 1] + off])
                return even + odd

            acc_a = seg_sum(0, _SEG_A)
            acc_b = seg_sum(_SEG_A, _SEG_B)
            ocol = pair_out + tp * 4
            plsc.store_scatter(outv, [jvec, row_idx, ocol], acc_a * inv_a)
            plsc.store_scatter(outv, [jvec, row_idx, ocol + 1], acc_b * inv_b)
            return carry
        lax.fori_loop(0, _CHUNK_T // 2, pair, 0)

    pltpu.make_async_copy(chunk_slice(0), buf0, sem0).start()
    pltpu.make_async_copy(chunk_slice(1), buf1, sem1).start()

    def tbody(t, carry):
        for p in range(2):
            j = 2 * t + p
            pltpu.make_async_copy(chunk_slice(j), bufs[p], sems[p]).wait()
            compute(bufs[p], j)

            @pl.when(j + 2 < _CHUNKS_PER_W)
            def _():
                pltpu.make_async_copy(
                    chunk_slice(j + 2), bufs[p], sems[p]).start()
        return carry

    lax.fori_loop(0, _CHUNKS_PER_W // 2, tbody, 0)

    fired = []
    for j in range(_CHUNKS_PER_W):
        c, tr, cb = chunk_coords(j)
        obase = (c * 16 + tr * 2 + cb // 8) * 1024 + (cb % 8) * 16
        for r in range(8):
            cp = pltpu.make_async_copy(
                outv.at[j, r], o_hbm.at[pl.ds(obase + r * 128, 16)], osem)
            cp.start()
            fired.append(cp)
    for cp in fired:
        cp.wait()


def kernel(inputs):
    xt = jnp.transpose(inputs, (2, 0, 1))
    mesh = plsc.VectorSubcoreMesh(core_axis_name="c", subcore_axis_name="s")
    run = pl.kernel(
        _sc_body,
        out_type=jax.ShapeDtypeStruct((_TILES // _B * 1024,), jnp.float32),
        mesh=mesh,
        scratch_types=[
            pltpu.VMEM((8, 1024), jnp.float32),
            pltpu.VMEM((8, 1024), jnp.float32),
            pltpu.VMEM((_CHUNKS_PER_W, 8, 16), jnp.float32),
            pltpu.SemaphoreType.DMA,
            pltpu.SemaphoreType.DMA,
            pltpu.SemaphoreType.DMA,
        ],
        compiler_params=pltpu.CompilerParams(needs_layout_passes=False),
    )
    out = run(xt)
    out = out.reshape(3, _B // 8, 2, 8, 128)
    out = out.transpose(1, 3, 2, 4, 0)
    return out.reshape(_B, 2 * _PERIODS, 3)


# speedup vs baseline: 1.9085x; 1.9085x over previous
import jax
import jax.numpy as jnp
from jax import lax
from jax.experimental import pallas as pl
from jax.experimental.pallas import tpu as pltpu
from jax.experimental.pallas import tpu_sc as plsc

_B = 64
_PERIODS = 128
_SEG_A = 48
_SEG_B = 80
_NW = 32
_TILES = 3 * (_B // 8) * _PERIODS
_CHUNK_T = 8
_CHUNKS = _TILES // _CHUNK_T
_CHUNKS_PER_W = _CHUNKS // _NW


def _sc_body(x_hbm, o_hbm, buf0, buf1, outv, cs, sem0, sem1, osem):
    cid = lax.axis_index("c")
    sid = lax.axis_index("s")
    w = sid * 2 + cid

    iota = lax.iota(jnp.int32, 16)
    inv = jnp.where(iota % 2 == 0, jnp.float32(1.0 / _SEG_A),
                    jnp.float32(1.0 / _SEG_B))

    bufs = (buf0, buf1)
    sems = (sem0, sem1)

    def chunk_coords(j):
        kg = w * _CHUNKS_PER_W + j
        c = kg // 128
        rem = kg - c * 128
        tr = rem // 16
        cb = rem - tr * 16
        return c, tr, cb

    def chunk_slice(j):
        c, tr, cb = chunk_coords(j)
        return x_hbm.at[c, pl.ds(tr * 8, 8), pl.ds(cb * 1024, 1024)]

    def compute(buf, j):
        def row(r, carry):
            for t in range(_CHUNK_T):
                base = t * 128
                va = (buf[r, pl.ds(base, 16)]
                      + buf[r, pl.ds(base + 16, 16)]
                      + buf[r, pl.ds(base + 32, 16)])
                vb = (buf[r, pl.ds(base + 48, 16)]
                      + buf[r, pl.ds(base + 64, 16)]
                      + buf[r, pl.ds(base + 80, 16)]
                      + buf[r, pl.ds(base + 96, 16)]
                      + buf[r, pl.ds(base + 112, 16)])
                cs[pl.ds(r * 256 + (2 * t) * 16, 16)] = jnp.cumsum(va)
                cs[pl.ds(r * 256 + (2 * t + 1) * 16, 16)] = jnp.cumsum(vb)
            totals = plsc.load_gather(cs, [r * 256 + iota * 16 + 15])
            outv[j, r] = totals * inv
            return carry
        lax.fori_loop(0, 8, row, 0)

    pltpu.make_async_copy(chunk_slice(0), buf0, sem0).start()
    pltpu.make_async_copy(chunk_slice(1), buf1, sem1).start()

    def tbody(t, carry):
        for p in range(2):
            j = 2 * t + p
            pltpu.make_async_copy(chunk_slice(j), bufs[p], sems[p]).wait()
            compute(bufs[p], j)

            @pl.when(j + 2 < _CHUNKS_PER_W)
            def _():
                pltpu.make_async_copy(
                    chunk_slice(j + 2), bufs[p], sems[p]).start()
        return carry

    lax.fori_loop(0, _CHUNKS_PER_W // 2, tbody, 0)

    fired = []
    for j in range(_CHUNKS_PER_W):
        c, tr, cb = chunk_coords(j)
        obase = (c * 16 + tr * 2 + cb // 8) * 1024 + (cb % 8) * 16
        for r in range(8):
            cp = pltpu.make_async_copy(
                outv.at[j, r], o_hbm.at[pl.ds(obase + r * 128, 16)], osem)
            cp.start()
            fired.append(cp)
    for cp in fired:
        cp.wait()


def kernel(inputs):
    xt = jnp.transpose(inputs, (2, 0, 1))
    mesh = plsc.VectorSubcoreMesh(core_axis_name="c", subcore_axis_name="s")
    run = pl.kernel(
        _sc_body,
        out_type=jax.ShapeDtypeStruct((_TILES // _B * 1024,), jnp.float32),
        mesh=mesh,
        scratch_types=[
            pltpu.VMEM((8, 1024), jnp.float32),
            pltpu.VMEM((8, 1024), jnp.float32),
            pltpu.VMEM((_CHUNKS_PER_W, 8, 16), jnp.float32),
            pltpu.VMEM((2048,), jnp.float32),
            pltpu.SemaphoreType.DMA,
            pltpu.SemaphoreType.DMA,
            pltpu.SemaphoreType.DMA,
        ],
        compiler_params=pltpu.CompilerParams(needs_layout_passes=False),
    )
    out = run(xt)
    out = out.reshape(3, _B // 8, 2, 8, 128)
    out = out.transpose(1, 3, 2, 4, 0)
    return out.reshape(_B, 2 * _PERIODS, 3)
